# baseline (speedup 1.0000x reference)
import functools

import jax
import jax.numpy as jnp
from jax import lax
from jax.experimental import pallas as pl
from jax.experimental.pallas import tpu as pltpu
from jax.experimental.pallas import tpu_sc as plsc

BATCH = 16384
D = 64
NC = 2
NS = 16
NW = NC * NS
BPW = BATCH // NW
CHUNK = 128
NCHUNK = BPW // CHUNK
GPC = CHUNK // 16


def _body(user_hbm, ii_hbm, ij_hbm, eu_hbm, ei_hbm, out_i_hbm, out_j_hbm,
          uidx_v, iidx_v, jidx_v, urow_v, irow_v, jrow_v,
          ua_v, ub_v, via_v, vib_v, vja_v, vjb_v,
          outi_v, outj_v, sems):
    wid = lax.axis_index("s") * NC + lax.axis_index("c")
    base = wid * BPW

    for c in range(NCHUNK):
        off = base + c * CHUNK
        pltpu.sync_copy(user_hbm.at[pl.ds(off, CHUNK)], uidx_v.at[c])
        pltpu.sync_copy(ii_hbm.at[pl.ds(off, CHUNK)], iidx_v.at[c])
        pltpu.sync_copy(ij_hbm.at[pl.ds(off, CHUNK)], jidx_v.at[c])

    for c in range(NCHUNK):
        for t in range(CHUNK // 16):
            s = pl.ds(t * 16, 16)
            urow_v[c, s] = lax.shift_right_logical(uidx_v[c, s], 1)
            irow_v[c, s] = lax.shift_right_logical(iidx_v[c, s], 1)
            jrow_v[c, s] = lax.shift_right_logical(jidx_v[c, s], 1)

    bufs = [(ua_v, via_v, vja_v), (ub_v, vib_v, vjb_v)]

    def start(q):
        ub, vib, vjb = bufs[q % 2]
        return [
            pltpu.async_copy(eu_hbm.at[urow_v.at[q]], ub, sems.at[0]),
            pltpu.async_copy(ei_hbm.at[irow_v.at[q]], vib, sems.at[1]),
            pltpu.async_copy(ei_hbm.at[jrow_v.at[q]], vjb, sems.at[2]),
        ]

    lanes = lax.iota(jnp.int32, 16)
    pending = start(0)

    for q in range(NCHUNK):
        for cp in pending:
            cp.wait()
        if q + 1 < NCHUNK:
            nxt = start(q + 1)
        ub, vib, vjb = bufs[q % 2]

        def group(g, carry, q=q, ub=ub, vib=vib, vjb=vjb):
            gb = g * 16
            rows = lanes + gb
            uoff = (uidx_v[q, pl.ds(gb, 16)] & 1) * D
            ioff = (iidx_v[q, pl.ds(gb, 16)] & 1) * D
            joff = (jidx_v[q, pl.ds(gb, 16)] & 1) * D
            acc_i = jnp.zeros((16,), jnp.float32)
            acc_j = jnp.zeros((16,), jnp.float32)
            for d in range(D):
                gu = plsc.load_gather(ub, [rows, uoff + d])
                gi = plsc.load_gather(vib, [rows, ioff + d])
                gj = plsc.load_gather(vjb, [rows, joff + d])
                acc_i = acc_i + gu * gi
                acc_j = acc_j + gu * gj
            outi_v[pl.ds(q * CHUNK + gb, 16)] = acc_i
            outj_v[pl.ds(q * CHUNK + gb, 16)] = acc_j
            return carry

        lax.fori_loop(0, GPC, group, 0)
        if q + 1 < NCHUNK:
            pending = nxt

    pltpu.sync_copy(outi_v, out_i_hbm.at[pl.ds(base, BPW)])
    pltpu.sync_copy(outj_v, out_j_hbm.at[pl.ds(base, BPW)])


@jax.jit
def _mf_forward(user, item_i, item_j, embed_user, embed_item):
    eu2 = embed_user.reshape(embed_user.shape[0] // 2, 2 * D)
    ei2 = embed_item.reshape(embed_item.shape[0] // 2, 2 * D)
    mesh = plsc.VectorSubcoreMesh(core_axis_name="c", subcore_axis_name="s")
    f = functools.partial(
        pl.kernel,
        out_type=(jax.ShapeDtypeStruct((BATCH,), jnp.float32),
                  jax.ShapeDtypeStruct((BATCH,), jnp.float32)),
        mesh=mesh,
        scratch_types=[
            pltpu.VMEM((NCHUNK, CHUNK), jnp.int32),
            pltpu.VMEM((NCHUNK, CHUNK), jnp.int32),
            pltpu.VMEM((NCHUNK, CHUNK), jnp.int32),
            pltpu.VMEM((NCHUNK, CHUNK), jnp.int32),
            pltpu.VMEM((NCHUNK, CHUNK), jnp.int32),
            pltpu.VMEM((NCHUNK, CHUNK), jnp.int32),
            pltpu.VMEM((CHUNK, 2 * D), jnp.float32),
            pltpu.VMEM((CHUNK, 2 * D), jnp.float32),
            pltpu.VMEM((CHUNK, 2 * D), jnp.float32),
            pltpu.VMEM((CHUNK, 2 * D), jnp.float32),
            pltpu.VMEM((CHUNK, 2 * D), jnp.float32),
            pltpu.VMEM((CHUNK, 2 * D), jnp.float32),
            pltpu.VMEM((BPW,), jnp.float32),
            pltpu.VMEM((BPW,), jnp.float32),
            pltpu.SemaphoreType.DMA((3,)),
        ],
        compiler_params=pltpu.CompilerParams(needs_layout_passes=False),
    )(_body)
    return f(user, item_i, item_j, eu2, ei2)


def kernel(user, item_i, item_j, embed_user, embed_item):
    return _mf_forward(user, item_i, item_j, embed_user, embed_item)


# speedup vs baseline: 1.4575x; 1.4575x over previous
import functools

import jax
import jax.numpy as jnp
from jax import lax
from jax.experimental import pallas as pl
from jax.experimental.pallas import tpu as pltpu
from jax.experimental.pallas import tpu_sc as plsc

BATCH = 16384
D = 64
NC = 2
NS = 16
NW = NC * NS
BPW = BATCH // NW
CHUNK = 128
NCHUNK = BPW // CHUNK
GPC = CHUNK // 16


def _lane(vec, s):
    return lax.squeeze(lax.slice(vec, (s,), (s + 1,)), (0,))


def _body(user_hbm, ii_hbm, ij_hbm, eu_hbm, ei_hbm, out_i_hbm, out_j_hbm,
          uidx_v, iidx_v, jidx_v, rows_a, outi_v, outj_v, sems):
    wid = lax.axis_index("s") * NC + lax.axis_index("c")
    base = wid * BPW

    for c in range(NCHUNK):
        off = base + c * CHUNK
        pltpu.sync_copy(user_hbm.at[pl.ds(off, CHUNK)], uidx_v.at[c])
        pltpu.sync_copy(ii_hbm.at[pl.ds(off, CHUNK)], iidx_v.at[c])
        pltpu.sync_copy(ij_hbm.at[pl.ds(off, CHUNK)], jidx_v.at[c])

    lanes = lax.iota(jnp.int32, 16)
    buf = rows_a

    def group(n, carry):
        q = lax.shift_right_logical(n, 3)
        gb = (n & 7) * 16
        iu = uidx_v[q, pl.ds(gb, 16)]
        ii = iidx_v[q, pl.ds(gb, 16)]
        ij = jidx_v[q, pl.ds(gb, 16)]
        tu = lax.shift_right_logical(iu, 3)
        ti = lax.shift_right_logical(ii, 3)
        tj = lax.shift_right_logical(ij, 3)
        cps = []
        for s in range(16):
            ru = pl.multiple_of(_lane(tu, s) * 8, 8)
            ri = pl.multiple_of(_lane(ti, s) * 8, 8)
            rj = pl.multiple_of(_lane(tj, s) * 8, 8)
            cps.append(pltpu.async_copy(
                eu_hbm.at[pl.ds(ru, 8), :], buf.at[s], sems.at[0]))
            cps.append(pltpu.async_copy(
                ei_hbm.at[pl.ds(ri, 8), :], buf.at[16 + s], sems.at[1]))
            cps.append(pltpu.async_copy(
                ei_hbm.at[pl.ds(rj, 8), :], buf.at[32 + s], sems.at[2]))
        for cp in cps:
            cp.wait()

        su = iu & 7
        si = ii & 7
        sj = ij & 7
        acc_i = jnp.zeros((16,), jnp.float32)
        acc_j = jnp.zeros((16,), jnp.float32)
        for s in range(16):
            ru = _lane(su, s)
            ri = _lane(si, s)
            rj = _lane(sj, s)
            u = [buf[s, ru, pl.ds(k * 16, 16)] for k in range(4)]
            vi = [buf[16 + s, ri, pl.ds(k * 16, 16)] for k in range(4)]
            vj = [buf[32 + s, rj, pl.ds(k * 16, 16)] for k in range(4)]
            pi = u[0] * vi[0] + u[1] * vi[1] + u[2] * vi[2] + u[3] * vi[3]
            pj = u[0] * vj[0] + u[1] * vj[1] + u[2] * vj[2] + u[3] * vj[3]
            for sh in (8, 4, 2, 1):
                pi = pi + pi.at[lanes ^ sh].get(
                    mode="promise_in_bounds", unique_indices=True)
                pj = pj + pj.at[lanes ^ sh].get(
                    mode="promise_in_bounds", unique_indices=True)
            acc_i = jnp.where(lanes == s, pi, acc_i)
            acc_j = jnp.where(lanes == s, pj, acc_j)
        outi_v[pl.ds(q * CHUNK + gb, 16)] = acc_i
        outj_v[pl.ds(q * CHUNK + gb, 16)] = acc_j
        return carry

    lax.fori_loop(0, NCHUNK * GPC, group, 0)

    pltpu.sync_copy(outi_v, out_i_hbm.at[pl.ds(base, BPW)])
    pltpu.sync_copy(outj_v, out_j_hbm.at[pl.ds(base, BPW)])


@jax.jit
def _mf_forward(user, item_i, item_j, embed_user, embed_item):
    mesh = plsc.VectorSubcoreMesh(core_axis_name="c", subcore_axis_name="s")
    f = functools.partial(
        pl.kernel,
        out_type=(jax.ShapeDtypeStruct((BATCH,), jnp.float32),
                  jax.ShapeDtypeStruct((BATCH,), jnp.float32)),
        mesh=mesh,
        scratch_types=[
            pltpu.VMEM((NCHUNK, CHUNK), jnp.int32),
            pltpu.VMEM((NCHUNK, CHUNK), jnp.int32),
            pltpu.VMEM((NCHUNK, CHUNK), jnp.int32),
            pltpu.VMEM((48, 8, D), jnp.float32),
            pltpu.VMEM((BPW,), jnp.float32),
            pltpu.VMEM((BPW,), jnp.float32),
            pltpu.SemaphoreType.DMA((3,)),
        ],
    )(_body)
    return f(user, item_i, item_j, embed_user, embed_item)


def kernel(user, item_i, item_j, embed_user, embed_item):
    return _mf_forward(user, item_i, item_j, embed_user, embed_item)


# speedup vs baseline: 2.2945x; 1.5743x over previous
import functools

import jax
import jax.numpy as jnp
from jax import lax
from jax.experimental import pallas as pl
from jax.experimental.pallas import tpu as pltpu
from jax.experimental.pallas import tpu_sc as plsc

BATCH = 16384
D = 64
NC = 2
NS = 16
NW = NC * NS
BPW = BATCH // NW
CHUNK = 128
NCHUNK = BPW // CHUNK
NGROUP = BPW // 16
SLAB = BPW // 2


def _lane(vec, s):
    return lax.squeeze(lax.slice(vec, (s,), (s + 1,)), (0,))


def _ubody(user_hbm, ett_hbm, slab_hbm,
           uidx_v, c0, c1, c2, c3, c4, c5, c6, c7, slab_v, sems):
    wid = lax.axis_index("s") * NC + lax.axis_index("c")
    base = wid * BPW
    for c in range(NCHUNK):
        pltpu.sync_copy(user_hbm.at[pl.ds(base + c * CHUNK, CHUNK)],
                        uidx_v.at[c])
    lanes = lax.iota(jnp.int32, 16)
    bufs = [c0, c1, c2, c3, c4, c5, c6, c7]

    def fire(r, b):
        bc = pl.multiple_of(lax.shift_right_logical(r, 7) * 128, 128)
        pltpu.async_copy(ett_hbm.at[:, pl.ds(bc, 128)], bufs[b], sems.at[b])

    v0 = uidx_v[0, pl.ds(0, 16)]
    for b in range(8):
        fire(_lane(v0, b), b)

    def group(g, carry):
        q = lax.shift_right_logical(g, 3)
        gb = (g & 7) * 16
        vg = uidx_v[q, pl.ds(gb, 16)]
        g1 = g + 1
        q1 = lax.shift_right_logical(g1, 3) & (NCHUNK - 1)
        vg1 = uidx_v[q1, pl.ds((g1 & 7) * 16, 16)]
        for s in range(16):
            b = s % 8
            buf = bufs[b]
            pltpu.make_async_copy(
                ett_hbm.at[:, pl.ds(0, 128)], buf, sems.at[b]).wait()
            r = _lane(vg, s)
            ca = (lax.shift_right_logical(r, 4) & 7) * 16
            clv = jnp.broadcast_to(r & 15, (16,))
            chunks = [jnp.zeros((16,), jnp.float32) for _ in range(4)]
            for d in range(D):
                ch = buf[d, pl.ds(ca, 16)]
                sp = ch.at[clv].get(mode="promise_in_bounds")
                k = d >> 4
                chunks[k] = jnp.where(lanes == (d & 15), sp, chunks[k])
            row = 8 * g + (s >> 1)
            half = (s & 1) * D
            for k in range(4):
                slab_v[row, pl.ds(half + k * 16, 16)] = chunks[k]
            nxt = 16 * g + s + 8

            @pl.when(nxt < BPW)
            def _(s=s, b=b, vg=vg, vg1=vg1):
                vv = vg if s < 8 else vg1
                fire(_lane(vv, (s + 8) % 16), b)
        return carry

    lax.fori_loop(0, NGROUP, group, 0)
    pltpu.sync_copy(slab_v, slab_hbm.at[pl.ds(wid * SLAB, SLAB), :])


def _ibody(ii_hbm, ij_hbm, ei_hbm, slab_hbm, out_i_hbm, out_j_hbm,
           iidx_v, jidx_v, uslab_v, tiles_a, tiles_b, outi_v, outj_v, sems):
    wid = lax.axis_index("s") * NC + lax.axis_index("c")
    base = wid * BPW
    for c in range(NCHUNK):
        pltpu.sync_copy(ii_hbm.at[pl.ds(base + c * CHUNK, CHUNK)],
                        iidx_v.at[c])
        pltpu.sync_copy(ij_hbm.at[pl.ds(base + c * CHUNK, CHUNK)],
                        jidx_v.at[c])
    pltpu.sync_copy(slab_hbm.at[pl.ds(wid * SLAB, SLAB), :], uslab_v)
    lanes = lax.iota(jnp.int32, 16)
    sets = [(tiles_a, 0, 1), (tiles_b, 2, 3)]

    def fire(g, which):
        tiles_v, s_i, s_j = sets[which]
        q = lax.shift_right_logical(g, 3)
        gb = (g & 7) * 16
        ti = lax.shift_right_logical(iidx_v[q, pl.ds(gb, 16)], 3)
        tj = lax.shift_right_logical(jidx_v[q, pl.ds(gb, 16)], 3)
        for s in range(16):
            ri = pl.multiple_of(_lane(ti, s) * 8, 8)
            rj = pl.multiple_of(_lane(tj, s) * 8, 8)
            pltpu.async_copy(
                ei_hbm.at[pl.ds(ri, 8), :], tiles_v.at[s], sems.at[s_i])
            pltpu.async_copy(
                ei_hbm.at[pl.ds(rj, 8), :], tiles_v.at[16 + s], sems.at[s_j])

    def compute(g, which):
        tiles_v, s_i, s_j = sets[which]
        for s in range(16):
            pltpu.make_async_copy(
                ei_hbm.at[pl.ds(0, 8), :], tiles_v.at[s],
                sems.at[s_i]).wait()
            pltpu.make_async_copy(
                ei_hbm.at[pl.ds(0, 8), :], tiles_v.at[16 + s],
                sems.at[s_j]).wait()
        q = lax.shift_right_logical(g, 3)
        gb = (g & 7) * 16
        ii = iidx_v[q, pl.ds(gb, 16)]
        ij = jidx_v[q, pl.ds(gb, 16)]
        si = ii & 7
        sj = ij & 7
        acc_i = jnp.zeros((16,), jnp.float32)
        acc_j = jnp.zeros((16,), jnp.float32)
        for s in range(16):
            ri = _lane(si, s)
            rj = _lane(sj, s)
            urow = 8 * g + (s >> 1)
            uh = (s & 1) * D
            u = [uslab_v[urow, pl.ds(uh + k * 16, 16)] for k in range(4)]
            vi = [tiles_v[s, ri, pl.ds(k * 16, 16)] for k in range(4)]
            vj = [tiles_v[16 + s, rj, pl.ds(k * 16, 16)] for k in range(4)]
            pi = u[0] * vi[0] + u[1] * vi[1] + u[2] * vi[2] + u[3] * vi[3]
            pj = u[0] * vj[0] + u[1] * vj[1] + u[2] * vj[2] + u[3] * vj[3]
            for sh in (8, 4, 2, 1):
                pi = pi + pi.at[lanes ^ sh].get(
                    mode="promise_in_bounds", unique_indices=True)
                pj = pj + pj.at[lanes ^ sh].get(
                    mode="promise_in_bounds", unique_indices=True)
            acc_i = jnp.where(lanes == s, pi, acc_i)
            acc_j = jnp.where(lanes == s, pj, acc_j)
        outi_v[pl.ds(g * 16, 16)] = acc_i
        outj_v[pl.ds(g * 16, 16)] = acc_j

    fire(0, 0)

    def pair(m, carry):
        g0 = m * 2
        fire(g0 + 1, 1)
        compute(g0, 0)

        @pl.when(m < NGROUP // 2 - 1)
        def _():
            fire(g0 + 2, 0)
        compute(g0 + 1, 1)
        return carry

    lax.fori_loop(0, NGROUP // 2, pair, 0)
    pltpu.sync_copy(outi_v, out_i_hbm.at[pl.ds(base, BPW)])
    pltpu.sync_copy(outj_v, out_j_hbm.at[pl.ds(base, BPW)])


@jax.jit
def _mf_forward(user, item_i, item_j, embed_user, embed_item):
    mesh = plsc.VectorSubcoreMesh(core_axis_name="c", subcore_axis_name="s")
    ett_u = embed_user.T

    k_user = functools.partial(
        pl.kernel,
        out_type=jax.ShapeDtypeStruct((BATCH // 2, 2 * D), jnp.float32),
        mesh=mesh,
        scratch_types=[
            pltpu.VMEM((NCHUNK, CHUNK), jnp.int32),
            pltpu.VMEM((D, 128), jnp.float32),
            pltpu.VMEM((D, 128), jnp.float32),
            pltpu.VMEM((D, 128), jnp.float32),
            pltpu.VMEM((D, 128), jnp.float32),
            pltpu.VMEM((D, 128), jnp.float32),
            pltpu.VMEM((D, 128), jnp.float32),
            pltpu.VMEM((D, 128), jnp.float32),
            pltpu.VMEM((D, 128), jnp.float32),
            pltpu.VMEM((SLAB, 2 * D), jnp.float32),
            pltpu.SemaphoreType.DMA((8,)),
        ],
    )(_ubody)
    u_slab = k_user(user, ett_u)

    k_item = functools.partial(
        pl.kernel,
        out_type=(jax.ShapeDtypeStruct((BATCH,), jnp.float32),
                  jax.ShapeDtypeStruct((BATCH,), jnp.float32)),
        mesh=mesh,
        scratch_types=[
            pltpu.VMEM((NCHUNK, CHUNK), jnp.int32),
            pltpu.VMEM((NCHUNK, CHUNK), jnp.int32),
            pltpu.VMEM((SLAB, 2 * D), jnp.float32),
            pltpu.VMEM((32, 8, D), jnp.float32),
            pltpu.VMEM((32, 8, D), jnp.float32),
            pltpu.VMEM((BPW,), jnp.float32),
            pltpu.VMEM((BPW,), jnp.float32),
            pltpu.SemaphoreType.DMA((4,)),
        ],
    )(_ibody)
    return k_item(item_i, item_j, embed_item, u_slab)


def kernel(user, item_i, item_j, embed_user, embed_item):
    return _mf_forward(user, item_i, item_j, embed_user, embed_item)


# speedup vs baseline: 2.3031x; 1.0038x over previous
import functools

import jax
import jax.numpy as jnp
from jax import lax
from jax.experimental import pallas as pl
from jax.experimental.pallas import tpu as pltpu
from jax.experimental.pallas import tpu_sc as plsc

BATCH = 16384
D = 64
NC = 2
NS = 16
NW = NC * NS
BPW = BATCH // NW
CHUNK = 128
NCHUNK = BPW // CHUNK
NGROUP = BPW // 16
SLAB = BPW // 2


def _lane(vec, s):
    return lax.squeeze(lax.slice(vec, (s,), (s + 1,)), (0,))


def _ubody(user_hbm, ett_hbm, slab_hbm,
           uidx_v, c0, c1, c2, c3, slab_v, sems):
    wid = lax.axis_index("s") * NC + lax.axis_index("c")
    base = wid * BPW
    for c in range(NCHUNK):
        pltpu.sync_copy(user_hbm.at[pl.ds(base + c * CHUNK, CHUNK)],
                        uidx_v.at[c])
    lanes = lax.iota(jnp.int32, 16)
    bufs = [c0, c1, c2, c3]

    def fire(r, b):
        bc = pl.multiple_of(lax.shift_right_logical(r, 7) * 128, 128)
        pltpu.async_copy(ett_hbm.at[:, pl.ds(bc, 128)], bufs[b], sems.at[b])

    v0 = uidx_v[0, pl.ds(0, 16)]
    for b in range(4):
        fire(_lane(v0, b), b)

    def group(g, carry):
        q = lax.shift_right_logical(g, 3)
        gb = (g & 7) * 16
        vg = uidx_v[q, pl.ds(gb, 16)]
        g1 = g + 1
        q1 = lax.shift_right_logical(g1, 3) & (NCHUNK - 1)
        vg1 = uidx_v[q1, pl.ds((g1 & 7) * 16, 16)]
        for s in range(16):
            b = s % 4
            buf = bufs[b]
            pltpu.make_async_copy(
                ett_hbm.at[:, pl.ds(0, 128)], buf, sems.at[b]).wait()
            r = _lane(vg, s)
            ca = (lax.shift_right_logical(r, 4) & 7) * 16
            clv = jnp.broadcast_to(r & 15, (16,))
            chunks = [jnp.zeros((16,), jnp.float32) for _ in range(4)]
            for d in range(D):
                ch = buf[d, pl.ds(ca, 16)]
                sp = ch.at[clv].get(mode="promise_in_bounds")
                k = d >> 4
                chunks[k] = jnp.where(lanes == (d & 15), sp, chunks[k])
            row = 8 * g + (s >> 1)
            half = (s & 1) * D
            for k in range(4):
                slab_v[row, pl.ds(half + k * 16, 16)] = chunks[k]
            nxt = 16 * g + s + 4

            @pl.when(nxt < BPW)
            def _(s=s, b=b, vg=vg, vg1=vg1):
                vv = vg if s < 12 else vg1
                fire(_lane(vv, (s + 4) % 16), b)
        return carry

    lax.fori_loop(0, NGROUP, group, 0)
    pltpu.sync_copy(slab_v, slab_hbm.at[pl.ds(wid * SLAB, SLAB), :])


def _ibody(ii_hbm, ij_hbm, ei_hbm, slab_hbm, out_i_hbm, out_j_hbm,
           iidx_v, jidx_v, uslab_v, tiles_a, tiles_b, outi_v, outj_v, sems):
    wid = lax.axis_index("s") * NC + lax.axis_index("c")
    base = wid * BPW
    for c in range(NCHUNK):
        pltpu.sync_copy(ii_hbm.at[pl.ds(base + c * CHUNK, CHUNK)],
                        iidx_v.at[c])
        pltpu.sync_copy(ij_hbm.at[pl.ds(base + c * CHUNK, CHUNK)],
                        jidx_v.at[c])
    pltpu.sync_copy(slab_hbm.at[pl.ds(wid * SLAB, SLAB), :], uslab_v)
    lanes = lax.iota(jnp.int32, 16)
    sets = [(tiles_a, 0, 1), (tiles_b, 2, 3)]

    def fire(g, which):
        tiles_v, s_i, s_j = sets[which]
        q = lax.shift_right_logical(g, 3)
        gb = (g & 7) * 16
        ti = lax.shift_right_logical(iidx_v[q, pl.ds(gb, 16)], 3)
        tj = lax.shift_right_logical(jidx_v[q, pl.ds(gb, 16)], 3)
        for s in range(16):
            ri = pl.multiple_of(_lane(ti, s) * 8, 8)
            rj = pl.multiple_of(_lane(tj, s) * 8, 8)
            pltpu.async_copy(
                ei_hbm.at[pl.ds(ri, 8), :], tiles_v.at[s], sems.at[s_i])
            pltpu.async_copy(
                ei_hbm.at[pl.ds(rj, 8), :], tiles_v.at[16 + s], sems.at[s_j])

    def compute(g, which):
        tiles_v, s_i, s_j = sets[which]
        for s in range(16):
            pltpu.make_async_copy(
                ei_hbm.at[pl.ds(0, 8), :], tiles_v.at[s],
                sems.at[s_i]).wait()
            pltpu.make_async_copy(
                ei_hbm.at[pl.ds(0, 8), :], tiles_v.at[16 + s],
                sems.at[s_j]).wait()
        q = lax.shift_right_logical(g, 3)
        gb = (g & 7) * 16
        ii = iidx_v[q, pl.ds(gb, 16)]
        ij = jidx_v[q, pl.ds(gb, 16)]
        si = ii & 7
        sj = ij & 7
        acc_i = jnp.zeros((16,), jnp.float32)
        acc_j = jnp.zeros((16,), jnp.float32)
        for s in range(16):
            ri = _lane(si, s)
            rj = _lane(sj, s)
            urow = 8 * g + (s >> 1)
            uh = (s & 1) * D
            u = [uslab_v[urow, pl.ds(uh + k * 16, 16)] for k in range(4)]
            vi = [tiles_v[s, ri, pl.ds(k * 16, 16)] for k in range(4)]
            vj = [tiles_v[16 + s, rj, pl.ds(k * 16, 16)] for k in range(4)]
            pi = u[0] * vi[0] + u[1] * vi[1] + u[2] * vi[2] + u[3] * vi[3]
            pj = u[0] * vj[0] + u[1] * vj[1] + u[2] * vj[2] + u[3] * vj[3]
            for sh in (8, 4, 2, 1):
                pi = pi + pi.at[lanes ^ sh].get(
                    mode="promise_in_bounds", unique_indices=True)
                pj = pj + pj.at[lanes ^ sh].get(
                    mode="promise_in_bounds", unique_indices=True)
            acc_i = jnp.where(lanes == s, pi, acc_i)
            acc_j = jnp.where(lanes == s, pj, acc_j)
        outi_v[pl.ds(g * 16, 16)] = acc_i
        outj_v[pl.ds(g * 16, 16)] = acc_j

    fire(0, 0)

    def pair(m, carry):
        g0 = m * 2
        fire(g0 + 1, 1)
        compute(g0, 0)

        @pl.when(m < NGROUP // 2 - 1)
        def _():
            fire(g0 + 2, 0)
        compute(g0 + 1, 1)
        return carry

    lax.fori_loop(0, NGROUP // 2, pair, 0)
    pltpu.sync_copy(outi_v, out_i_hbm.at[pl.ds(base, BPW)])
    pltpu.sync_copy(outj_v, out_j_hbm.at[pl.ds(base, BPW)])


@jax.jit
def _mf_forward(user, item_i, item_j, embed_user, embed_item):
    mesh = plsc.VectorSubcoreMesh(core_axis_name="c", subcore_axis_name="s")
    ett_u = embed_user.T

    k_user = functools.partial(
        pl.kernel,
        out_type=jax.ShapeDtypeStruct((BATCH // 2, 2 * D), jnp.float32),
        mesh=mesh,
        scratch_types=[
            pltpu.VMEM((NCHUNK, CHUNK), jnp.int32),
            pltpu.VMEM((D, 128), jnp.float32),
            pltpu.VMEM((D, 128), jnp.float32),
            pltpu.VMEM((D, 128), jnp.float32),
            pltpu.VMEM((D, 128), jnp.float32),
            pltpu.VMEM((SLAB, 2 * D), jnp.float32),
            pltpu.SemaphoreType.DMA((4,)),
        ],
    )(_ubody)
    u_slab = k_user(user, ett_u)

    k_item = functools.partial(
        pl.kernel,
        out_type=(jax.ShapeDtypeStruct((BATCH,), jnp.float32),
                  jax.ShapeDtypeStruct((BATCH,), jnp.float32)),
        mesh=mesh,
        scratch_types=[
            pltpu.VMEM((NCHUNK, CHUNK), jnp.int32),
            pltpu.VMEM((NCHUNK, CHUNK), jnp.int32),
            pltpu.VMEM((SLAB, 2 * D), jnp.float32),
            pltpu.VMEM((32, 8, D), jnp.float32),
            pltpu.VMEM((32, 8, D), jnp.float32),
            pltpu.VMEM((BPW,), jnp.float32),
            pltpu.VMEM((BPW,), jnp.float32),
            pltpu.SemaphoreType.DMA((4,)),
        ],
    )(_ibody)
    return k_item(item_i, item_j, embed_item, u_slab)


def kernel(user, item_i, item_j, embed_user, embed_item):
    return _mf_forward(user, item_i, item_j, embed_user, embed_item)


# speedup vs baseline: 2.3181x; 1.0065x over previous
import functools

import jax
import jax.numpy as jnp
from jax import lax
from jax.experimental import pallas as pl
from jax.experimental.pallas import tpu as pltpu
from jax.experimental.pallas import tpu_sc as plsc

BATCH = 16384
D = 64
NC = 2
NS = 16
NW = NC * NS
BPW = BATCH // NW
CHUNK = 128
NCHUNK = BPW // CHUNK
NGROUP = BPW // 16
SLAB = BPW // 2


def _lane(vec, s):
    return lax.squeeze(lax.slice(vec, (s,), (s + 1,)), (0,))


def _ubody(user_hbm, ett_hbm, slab_hbm,
           uidx_v, c0, c1, c2, c3, slab_v, sems):
    wid = lax.axis_index("s") * NC + lax.axis_index("c")
    base = wid * BPW
    for c in range(NCHUNK):
        pltpu.sync_copy(user_hbm.at[pl.ds(base + c * CHUNK, CHUNK)],
                        uidx_v.at[c])
    lanes = lax.iota(jnp.int32, 16)
    bufs = [c0, c1, c2, c3]

    def fire(r, b):
        bc = pl.multiple_of(lax.shift_right_logical(r, 7) * 128, 128)
        pltpu.async_copy(ett_hbm.at[:, pl.ds(bc, 128)], bufs[b], sems.at[b])

    v0 = uidx_v[0, pl.ds(0, 16)]
    for b in range(4):
        fire(_lane(v0, b), b)

    def group(g, carry):
        q = lax.shift_right_logical(g, 3)
        gb = (g & 7) * 16
        vg = uidx_v[q, pl.ds(gb, 16)]
        g1 = g + 1
        q1 = lax.shift_right_logical(g1, 3) & (NCHUNK - 1)
        vg1 = uidx_v[q1, pl.ds((g1 & 7) * 16, 16)]
        for s in range(16):
            b = s % 4
            buf = bufs[b]
            pltpu.make_async_copy(
                ett_hbm.at[:, pl.ds(0, 128)], buf, sems.at[b]).wait()
            r = _lane(vg, s)
            ca = (lax.shift_right_logical(r, 4) & 7) * 16
            clv = jnp.broadcast_to(r & 15, (16,))
            chunks = [jnp.zeros((16,), jnp.float32) for _ in range(4)]
            for d in range(D):
                ch = buf[d, pl.ds(ca, 16)]
                sp = ch.at[clv].get(mode="promise_in_bounds")
                k = d >> 4
                chunks[k] = jnp.where(lanes == (d & 15), sp, chunks[k])
            row = 8 * g + (s >> 1)
            half = (s & 1) * D
            for k in range(4):
                slab_v[row, pl.ds(half + k * 16, 16)] = chunks[k]
            nxt = 16 * g + s + 4

            @pl.when(nxt < BPW)
            def _(s=s, b=b, vg=vg, vg1=vg1):
                vv = vg if s < 12 else vg1
                fire(_lane(vv, (s + 4) % 16), b)
        return carry

    lax.fori_loop(0, NGROUP, group, 0)
    pltpu.sync_copy(slab_v, slab_hbm.at[pl.ds(wid * SLAB, SLAB), :])


def _ibody(ii_hbm, ij_hbm, ei_hbm, slab_hbm, out_i_hbm, out_j_hbm,
           iidx_v, jidx_v, uslab_v, tiles_a, tiles_b, outi_v, outj_v, sems):
    wid = lax.axis_index("s") * NC + lax.axis_index("c")
    base = wid * BPW
    stage = [pltpu.async_copy(slab_hbm.at[pl.ds(wid * SLAB, SLAB), :],
                              uslab_v, sems.at[0])]
    for c in range(NCHUNK):
        stage.append(pltpu.async_copy(
            ii_hbm.at[pl.ds(base + c * CHUNK, CHUNK)], iidx_v.at[c],
            sems.at[0]))
        stage.append(pltpu.async_copy(
            ij_hbm.at[pl.ds(base + c * CHUNK, CHUNK)], jidx_v.at[c],
            sems.at[0]))
    for cp in stage:
        cp.wait()
    lanes = lax.iota(jnp.int32, 16)
    sets = [(tiles_a, 0, 1), (tiles_b, 2, 3)]

    def fire(g, which):
        tiles_v, s_i, s_j = sets[which]
        q = lax.shift_right_logical(g, 3)
        gb = (g & 7) * 16
        ti = lax.shift_right_logical(iidx_v[q, pl.ds(gb, 16)], 3)
        tj = lax.shift_right_logical(jidx_v[q, pl.ds(gb, 16)], 3)
        for s in range(16):
            ri = pl.multiple_of(_lane(ti, s) * 8, 8)
            rj = pl.multiple_of(_lane(tj, s) * 8, 8)
            pltpu.async_copy(
                ei_hbm.at[pl.ds(ri, 8), :], tiles_v.at[s], sems.at[s_i])
            pltpu.async_copy(
                ei_hbm.at[pl.ds(rj, 8), :], tiles_v.at[16 + s], sems.at[s_j])

    def compute(g, which):
        tiles_v, s_i, s_j = sets[which]
        for s in range(16):
            pltpu.make_async_copy(
                ei_hbm.at[pl.ds(0, 8), :], tiles_v.at[s],
                sems.at[s_i]).wait()
            pltpu.make_async_copy(
                ei_hbm.at[pl.ds(0, 8), :], tiles_v.at[16 + s],
                sems.at[s_j]).wait()
        q = lax.shift_right_logical(g, 3)
        gb = (g & 7) * 16
        ii = iidx_v[q, pl.ds(gb, 16)]
        ij = jidx_v[q, pl.ds(gb, 16)]
        si = ii & 7
        sj = ij & 7
        acc_i = jnp.zeros((16,), jnp.float32)
        acc_j = jnp.zeros((16,), jnp.float32)
        for s in range(16):
            ri = _lane(si, s)
            rj = _lane(sj, s)
            urow = 8 * g + (s >> 1)
            uh = (s & 1) * D
            u = [uslab_v[urow, pl.ds(uh + k * 16, 16)] for k in range(4)]
            vi = [tiles_v[s, ri, pl.ds(k * 16, 16)] for k in range(4)]
            vj = [tiles_v[16 + s, rj, pl.ds(k * 16, 16)] for k in range(4)]
            pi = u[0] * vi[0] + u[1] * vi[1] + u[2] * vi[2] + u[3] * vi[3]
            pj = u[0] * vj[0] + u[1] * vj[1] + u[2] * vj[2] + u[3] * vj[3]
            for sh in (8, 4, 2, 1):
                pi = pi + pi.at[lanes ^ sh].get(
                    mode="promise_in_bounds", unique_indices=True)
                pj = pj + pj.at[lanes ^ sh].get(
                    mode="promise_in_bounds", unique_indices=True)
            acc_i = jnp.where(lanes == s, pi, acc_i)
            acc_j = jnp.where(lanes == s, pj, acc_j)
        outi_v[pl.ds(g * 16, 16)] = acc_i
        outj_v[pl.ds(g * 16, 16)] = acc_j

    fire(0, 0)

    def pair(m, carry):
        g0 = m * 2
        fire(g0 + 1, 1)
        compute(g0, 0)

        @pl.when(m < NGROUP // 2 - 1)
        def _():
            fire(g0 + 2, 0)
        compute(g0 + 1, 1)
        return carry

    lax.fori_loop(0, NGROUP // 2, pair, 0)
    pltpu.sync_copy(outi_v, out_i_hbm.at[pl.ds(base, BPW)])
    pltpu.sync_copy(outj_v, out_j_hbm.at[pl.ds(base, BPW)])


@jax.jit
def _mf_forward(user, item_i, item_j, embed_user, embed_item):
    mesh = plsc.VectorSubcoreMesh(core_axis_name="c", subcore_axis_name="s")
    ett_u = embed_user.T

    k_user = functools.partial(
        pl.kernel,
        out_type=jax.ShapeDtypeStruct((BATCH // 2, 2 * D), jnp.float32),
        mesh=mesh,
        scratch_types=[
            pltpu.VMEM((NCHUNK, CHUNK), jnp.int32),
            pltpu.VMEM((D, 128), jnp.float32),
            pltpu.VMEM((D, 128), jnp.float32),
            pltpu.VMEM((D, 128), jnp.float32),
            pltpu.VMEM((D, 128), jnp.float32),
            pltpu.VMEM((SLAB, 2 * D), jnp.float32),
            pltpu.SemaphoreType.DMA((4,)),
        ],
    )(_ubody)
    u_slab = k_user(user, ett_u)

    k_item = functools.partial(
        pl.kernel,
        out_type=(jax.ShapeDtypeStruct((BATCH,), jnp.float32),
                  jax.ShapeDtypeStruct((BATCH,), jnp.float32)),
        mesh=mesh,
        scratch_types=[
            pltpu.VMEM((NCHUNK, CHUNK), jnp.int32),
            pltpu.VMEM((NCHUNK, CHUNK), jnp.int32),
            pltpu.VMEM((SLAB, 2 * D), jnp.float32),
            pltpu.VMEM((32, 8, D), jnp.float32),
            pltpu.VMEM((32, 8, D), jnp.float32),
            pltpu.VMEM((BPW,), jnp.float32),
            pltpu.VMEM((BPW,), jnp.float32),
            pltpu.SemaphoreType.DMA((4,)),
        ],
    )(_ibody)
    return k_item(item_i, item_j, embed_item, u_slab)


def kernel(user, item_i, item_j, embed_user, embed_item):
    return _mf_forward(user, item_i, item_j, embed_user, embed_item)
